# Initial kernel scaffold; baseline (speedup 1.0000x reference)
import jax
import jax.numpy as jnp
from jax.experimental import pallas as pl


def kernel(x, edge_index, W1, W2):
    raise NotImplementedError("write your pallas kernel here")


# speedup vs baseline: 5.9029x; 5.9029x over previous
import functools

import jax
import jax.numpy as jnp
from jax import lax
from jax.experimental import pallas as pl
from jax.experimental.pallas import tpu as pltpu
from jax.experimental.pallas import tpu_sc as plsc

_K = 128
_NT = 16
_NC = 2
_NW = _NC * _NT


def _pre_body(x_ref, o_ref):
    x = x_ref[...]
    n, d = x.shape
    spars = 1.0 - jnp.sum((x != 0).astype(jnp.float32), axis=1, keepdims=True) / d
    ent = -jnp.sum(x * jnp.log(x + 1e-15), axis=1, keepdims=True)
    spars = (spars - jnp.min(spars)) / (jnp.max(spars) - jnp.min(spars))
    ent = (ent - jnp.min(ent)) / (jnp.max(ent) - jnp.min(ent))
    pad = jnp.zeros((n, o_ref.shape[1] - d - 2), jnp.float32)
    o_ref[...] = jnp.concatenate([x, spars, ent, pad], axis=1)


def _mid_body(n, p_ref, w1_ref, w2_ref, o_ref):
    a = p_ref[0, :n, :] + p_ref[1, :n, :]
    h = jnp.dot(a, w1_ref[...], preferred_element_type=jnp.float32,
                precision=lax.Precision.HIGHEST)
    h = jnp.maximum(h, 0.0)
    d = h.shape[1]
    spars = 1.0 - jnp.sum((h != 0).astype(jnp.float32), axis=1, keepdims=True) / d
    ent = -jnp.sum(h * jnp.log(h + 1e-15), axis=1, keepdims=True)
    spars = (spars - jnp.min(spars)) / (jnp.max(spars) - jnp.min(spars))
    ent = (ent - jnp.min(ent)) / (jnp.max(ent) - jnp.min(ent))
    y = jnp.dot(h, w2_ref[:d, :], preferred_element_type=jnp.float32,
                precision=lax.Precision.HIGHEST)
    y = y + spars * w2_ref[d:d + 1, :] + ent * w2_ref[d + 1:d + 2, :]
    o_ref[...] = y


def _post_body(n, p_ref, o_ref):
    a = p_ref[0, :n, :] + p_ref[1, :n, :]
    s = a - jnp.max(a, axis=1, keepdims=True)
    o_ref[...] = s - jnp.log(jnp.sum(jnp.exp(s), axis=1, keepdims=True))


def _make_agg(n_tab, np_, d, ch):
    rows_per_tile = np_ // _NT
    mesh = plsc.VectorSubcoreMesh(core_axis_name="c", subcore_axis_name="s")

    @functools.partial(
        pl.kernel,
        out_type=jax.ShapeDtypeStruct((_NC, np_, d), jnp.float32),
        mesh=mesh,
        scratch_types=[
            pltpu.VMEM((ch, _K), jnp.int32),
            pltpu.VMEM((ch, _K), jnp.int32),
            pltpu.VMEM((_K, d), jnp.float32),
            pltpu.VMEM_SHARED((np_, d), jnp.float32),
            pltpu.SemaphoreType.DMA,
        ],
        compiler_params=pltpu.CompilerParams(use_tc_tiling_on_sc=False),
    )
    def agg(table, rowc, colc, zeros, out, rowv, colv, buf, shared, sem):
        cid = lax.axis_index("c")
        sid = lax.axis_index("s")
        pltpu.sync_copy(rowc.at[cid, sid], rowv)
        pltpu.sync_copy(colc.at[cid, sid], colv)
        r0 = sid * rows_per_tile
        pltpu.sync_copy(zeros.at[pl.ds(r0, rows_per_tile)],
                        shared.at[pl.ds(r0, rows_per_tile)])
        plsc.subcore_barrier()

        def body(i, carry):
            pltpu.async_copy(table.at[rowv.at[i]], buf, sem).wait()
            pltpu.sync_copy(buf, shared.at[colv.at[i]], add=True)
            return carry

        lax.fori_loop(0, ch, body, 0)
        plsc.subcore_barrier()
        pltpu.sync_copy(shared.at[pl.ds(r0, rows_per_tile)],
                        out.at[cid, pl.ds(r0, rows_per_tile)])

    return agg


def kernel(x, edge_index, W1, W2):
    n, f_in = x.shape
    e = edge_index.shape[1]
    hid = W1.shape[1]
    cls = W2.shape[1]
    d1 = ((f_in + 2 + 15) // 16) * 16
    np_ = ((n + _NT * 8 - 1) // (_NT * 8)) * _NT * 8
    ch = (e + _NW * _K - 1) // (_NW * _K)
    e_pad = _NW * _K * ch

    row = edge_index[0]
    col = edge_index[1]
    pad = e_pad - e
    rowp = jnp.concatenate([row, jnp.zeros((pad,), jnp.int32)])
    colp = jnp.concatenate([col, jnp.full((pad,), n, jnp.int32)])
    rowc = rowp.reshape(_NC, _NT, ch, _K)
    colc = colp.reshape(_NC, _NT, ch, _K)
    w1p = jnp.zeros((d1, hid), jnp.float32).at[:f_in + 2, :].set(W1)
    zeros1 = jnp.zeros((np_, d1), jnp.float32)
    zeros2 = jnp.zeros((np_, cls), jnp.float32)

    xc1 = pl.pallas_call(
        _pre_body,
        out_shape=jax.ShapeDtypeStruct((n, d1), jnp.float32),
    )(x)
    part1 = _make_agg(n, np_, d1, ch)(xc1, rowc, colc, zeros1)
    y2 = pl.pallas_call(
        functools.partial(_mid_body, n),
        out_shape=jax.ShapeDtypeStruct((n, cls), jnp.float32),
        compiler_params=pltpu.CompilerParams(vmem_limit_bytes=100 * 1024 * 1024),
    )(part1, w1p, W2)

    part2 = _make_agg(n, np_, cls, ch)(y2, rowc, colc, zeros2)
    out = pl.pallas_call(
        functools.partial(_post_body, n),
        out_shape=jax.ShapeDtypeStruct((n, cls), jnp.float32),
    )(part2)
    return out
